# baseline (speedup 1.0000x reference)
import jax
import jax.numpy as jnp
from jax import lax
from jax.experimental import pallas as pl
from jax.experimental.pallas import tpu as pltpu
from jax.experimental.pallas import tpu_sc as plsc

_N = 10000
_E = 320000
_F_IN = 128
_HID = 256
_CLS = 64

_NC = 2
_NS = 16
_NW = _NC * _NS
_NP = 10240
_RPT = _NP // _NS
_D1 = 136

_CHUNK1 = 80
_NCHUNK1 = _E // _NW // _CHUNK1
_CHUNK2 = 128
_NCHUNK2 = 79
_EP2 = _NW * _NCHUNK2 * _CHUNK2


def _sc_segsum(table, row_r, col_r, zeros_rt, d, chunk, nchunk):
    mesh = plsc.VectorSubcoreMesh(core_axis_name="c", subcore_axis_name="s")

    def body(table_hbm, row_hbm, col_hbm, zero_hbm, out_hbm,
             row_v, col_v, buf_a, buf_b, acc, sem_a, sem_b):
        c = lax.axis_index("c")
        s = lax.axis_index("s")
        pltpu.sync_copy(zero_hbm, acc.at[pl.ds(s * _RPT, _RPT)])
        pltpu.sync_copy(row_hbm.at[c, s], row_v)
        pltpu.sync_copy(col_hbm.at[c, s], col_v)
        plsc.subcore_barrier()

        pltpu.async_copy(table_hbm.at[row_v.at[0]], buf_a, sem_a)
        pairs_end = nchunk - 1 if nchunk % 2 else nchunk

        @pl.loop(0, pairs_end, step=2)
        def pair(j):
            pltpu.async_copy(table_hbm.at[row_v.at[j + 1]], buf_b, sem_b)
            pltpu.make_async_copy(table_hbm.at[row_v.at[j]], buf_a, sem_a).wait()
            pltpu.sync_copy(buf_a, acc.at[col_v.at[j]], add=True)

            @pl.when(j + 2 < nchunk)
            def _():
                pltpu.async_copy(table_hbm.at[row_v.at[j + 2]], buf_a, sem_a)

            pltpu.make_async_copy(table_hbm.at[row_v.at[j + 1]], buf_b,
                                  sem_b).wait()
            pltpu.sync_copy(buf_b, acc.at[col_v.at[j + 1]], add=True)

        if nchunk % 2:
            pltpu.make_async_copy(table_hbm.at[row_v.at[nchunk - 1]], buf_a,
                                  sem_a).wait()
            pltpu.sync_copy(buf_a, acc.at[col_v.at[nchunk - 1]], add=True)

        plsc.subcore_barrier()
        pltpu.sync_copy(acc.at[pl.ds(s * _RPT, _RPT)],
                        out_hbm.at[c, pl.ds(s * _RPT, _RPT)])

    f = pl.kernel(
        body,
        out_type=jax.ShapeDtypeStruct((_NC, _NP, d), jnp.float32),
        mesh=mesh,
        scratch_types=[
            pltpu.VMEM((nchunk, chunk), jnp.int32),
            pltpu.VMEM((nchunk, chunk), jnp.int32),
            pltpu.VMEM((chunk, d), jnp.float32),
            pltpu.VMEM((chunk, d), jnp.float32),
            pltpu.VMEM_SHARED((_NP, d), jnp.float32),
            pltpu.SemaphoreType.DMA,
            pltpu.SemaphoreType.DMA,
        ],
        compiler_params=pltpu.CompilerParams(use_tc_tiling_on_sc=False),
    )
    return f(table, row_r, col_r, zeros_rt)


def _table1_body(x_ref, t_ref):
    x = x_ref[...]
    nnz = jnp.sum((x != 0.0).astype(jnp.float32), axis=1)
    fs = 1.0 - nnz / _F_IN
    fe = -jnp.sum(x * jnp.log(x + 1e-15), axis=1)
    fs = (fs - jnp.min(fs)) / (jnp.max(fs) - jnp.min(fs))
    fe = (fe - jnp.min(fe)) / (jnp.max(fe) - jnp.min(fe))
    pad = jnp.zeros((x.shape[0], _D1 - _F_IN - 2), jnp.float32)
    t_ref[...] = jnp.concatenate([x, fs[:, None], fe[:, None], pad], axis=1)


def _layer1_body(p_ref, w1_ref, w2a_ref, wse_ref, y_ref):
    aggr = p_ref[0, :_N] + p_ref[1, :_N]
    h = jnp.dot(aggr, w1_ref[...], preferred_element_type=jnp.float32)
    h = jnp.maximum(h, 0.0)
    nnz = jnp.sum((h != 0.0).astype(jnp.float32), axis=1)
    fs = 1.0 - nnz / _HID
    fe = -jnp.sum(h * jnp.log(h + 1e-15), axis=1)
    fs = (fs - jnp.min(fs)) / (jnp.max(fs) - jnp.min(fs))
    fe = (fe - jnp.min(fe)) / (jnp.max(fe) - jnp.min(fe))
    y = jnp.dot(h, w2a_ref[...], preferred_element_type=jnp.float32)
    y_ref[...] = (y + fs[:, None] * wse_ref[0][None, :]
                  + fe[:, None] * wse_ref[1][None, :])


def _lsm_body(p_ref, o_ref):
    z = p_ref[0, :_N] + p_ref[1, :_N]
    m = jnp.max(z, axis=1, keepdims=True)
    o_ref[...] = (z - m) - jnp.log(jnp.sum(jnp.exp(z - m), axis=1,
                                           keepdims=True))


def kernel(x, edge_index, W1, W2):
    row1 = edge_index[0].reshape(_NC, _NS, _NCHUNK1, _CHUNK1)
    col1 = edge_index[1].reshape(_NC, _NS, _NCHUNK1, _CHUNK1)
    pad_r = jnp.zeros((_EP2 - _E,), jnp.int32)
    pad_c = _N + jnp.arange(_EP2 - _E, dtype=jnp.int32) % (_NP - _N)
    row2 = jnp.concatenate([edge_index[0], pad_r]).reshape(
        _NC, _NS, _NCHUNK2, _CHUNK2)
    col2 = jnp.concatenate([edge_index[1], pad_c]).reshape(
        _NC, _NS, _NCHUNK2, _CHUNK2)
    zeros1 = jnp.zeros((_RPT, _D1), jnp.float32)
    zeros2 = jnp.zeros((_RPT, _CLS), jnp.float32)
    w1p = jnp.pad(W1, ((0, _D1 - (_F_IN + 2)), (0, 0)))
    w2a = W2[:_HID]
    wse = W2[_HID:_HID + 2]

    table1 = pl.pallas_call(
        _table1_body,
        out_shape=jax.ShapeDtypeStruct((_N, _D1), jnp.float32),
    )(x)
    p1 = _sc_segsum(table1, row1, col1, zeros1, _D1, _CHUNK1, _NCHUNK1)
    y = pl.pallas_call(
        _layer1_body,
        out_shape=jax.ShapeDtypeStruct((_N, _CLS), jnp.float32),
    )(p1, w1p, w2a, wse)
    p2 = _sc_segsum(y, row2, col2, zeros2, _CLS, _CHUNK2, _NCHUNK2)
    return pl.pallas_call(
        _lsm_body,
        out_shape=jax.ShapeDtypeStruct((_N, _CLS), jnp.float32),
    )(p2)


# speedup vs baseline: 1.2053x; 1.2053x over previous
import jax
import jax.numpy as jnp
from jax import lax
from jax.experimental import pallas as pl
from jax.experimental.pallas import tpu as pltpu
from jax.experimental.pallas import tpu_sc as plsc

_N = 10000
_E = 320000
_F_IN = 128
_HID = 256
_CLS = 64

_NC = 2
_NS = 16
_NW = _NC * _NS
_NP = 10240
_RPT = _NP // _NS
_D1 = 136

_CHUNK1 = 80
_NCHUNK1 = _E // _NW // _CHUNK1
_CHUNK2 = 128
_NCHUNK2 = 79
_EP2 = _NW * _NCHUNK2 * _CHUNK2


def _sc_segsum(table, row_r, col_r, zeros_rt, d, chunk, nchunk):
    mesh = plsc.VectorSubcoreMesh(core_axis_name="c", subcore_axis_name="s")

    def body(table_hbm, row_hbm, col_hbm, zero_hbm, out_hbm,
             row_v, col_v, buf_a, buf_b, acc, sem_a, sem_b):
        c = lax.axis_index("c")
        s = lax.axis_index("s")
        pltpu.sync_copy(zero_hbm, acc.at[pl.ds(s * _RPT, _RPT)])
        pltpu.sync_copy(row_hbm.at[c, s], row_v)
        pltpu.sync_copy(col_hbm.at[c, s], col_v)
        plsc.subcore_barrier()

        pltpu.async_copy(table_hbm.at[row_v.at[0]], buf_a, sem_a)
        pairs_end = nchunk - 1 if nchunk % 2 else nchunk

        @pl.loop(0, pairs_end, step=2)
        def pair(j):
            pltpu.async_copy(table_hbm.at[row_v.at[j + 1]], buf_b, sem_b)
            pltpu.make_async_copy(table_hbm.at[row_v.at[j]], buf_a, sem_a).wait()
            pltpu.sync_copy(buf_a, acc.at[col_v.at[j]], add=True)

            @pl.when(j + 2 < nchunk)
            def _():
                pltpu.async_copy(table_hbm.at[row_v.at[j + 2]], buf_a, sem_a)

            pltpu.make_async_copy(table_hbm.at[row_v.at[j + 1]], buf_b,
                                  sem_b).wait()
            pltpu.sync_copy(buf_b, acc.at[col_v.at[j + 1]], add=True)

        if nchunk % 2:
            pltpu.make_async_copy(table_hbm.at[row_v.at[nchunk - 1]], buf_a,
                                  sem_a).wait()
            pltpu.sync_copy(buf_a, acc.at[col_v.at[nchunk - 1]], add=True)

        plsc.subcore_barrier()
        pltpu.sync_copy(acc.at[pl.ds(s * _RPT, _RPT)],
                        out_hbm.at[c, pl.ds(s * _RPT, _RPT)])

    f = pl.kernel(
        body,
        out_type=jax.ShapeDtypeStruct((_NC, _NP, d), jnp.float32),
        mesh=mesh,
        scratch_types=[
            pltpu.VMEM((nchunk, chunk), jnp.int32),
            pltpu.VMEM((nchunk, chunk), jnp.int32),
            pltpu.VMEM((chunk, d), jnp.float32),
            pltpu.VMEM((chunk, d), jnp.float32),
            pltpu.VMEM_SHARED((_NP, d), jnp.float32),
            pltpu.SemaphoreType.DMA,
            pltpu.SemaphoreType.DMA,
        ],
        compiler_params=pltpu.CompilerParams(use_tc_tiling_on_sc=False),
    )
    return f(table, row_r, col_r, zeros_rt)


def _table1_body(x_ref, t_ref):
    x = x_ref[...]
    nnz = jnp.sum((x != 0.0).astype(jnp.float32), axis=1)
    fs = 1.0 - nnz / _F_IN
    fe = -jnp.sum(x * jnp.log(x + 1e-15), axis=1)
    fs = (fs - jnp.min(fs)) / (jnp.max(fs) - jnp.min(fs))
    fe = (fe - jnp.min(fe)) / (jnp.max(fe) - jnp.min(fe))
    pad = jnp.zeros((x.shape[0], _D1 - _F_IN - 2), jnp.float32)
    t_ref[...] = jnp.concatenate([x, fs[:, None], fe[:, None], pad], axis=1)


_NB = 10
_BB = _N // _NB


def _layer1_body(p_hbm, w1_ref, w2_ref, y_ref,
                 buf0, buf1, h_scr, fs_scr, fe_scr, sem0, sem1):
    bufs, sems = (buf0, buf1), (sem0, sem1)
    pltpu.async_copy(p_hbm.at[:, pl.ds(0, _BB)], buf0, sem0)
    for i in range(_NB):
        cur, csem = bufs[i % 2], sems[i % 2]
        if i + 1 < _NB:
            pltpu.async_copy(p_hbm.at[:, pl.ds((i + 1) * _BB, _BB)],
                             bufs[(i + 1) % 2], sems[(i + 1) % 2])
        pltpu.make_async_copy(p_hbm.at[:, pl.ds(i * _BB, _BB)], cur,
                              csem).wait()
        aggr = cur[0] + cur[1]
        hb = jnp.dot(aggr[:, :_F_IN + 2], w1_ref[...],
                     preferred_element_type=jnp.float32)
        hb = jnp.maximum(hb, 0.0)
        h_scr[pl.ds(i * _BB, _BB)] = hb
        nnz = jnp.sum((hb != 0.0).astype(jnp.float32), axis=1)
        fs_scr[i] = 1.0 - nnz / _HID
        fe_scr[i] = -jnp.sum(hb * jnp.log(hb + 1e-15), axis=1)
    fs_mn, fs_mx = jnp.min(fs_scr[...]), jnp.max(fs_scr[...])
    fe_mn, fe_mx = jnp.min(fe_scr[...]), jnp.max(fe_scr[...])
    for i in range(_NB):
        fs = (fs_scr[i] - fs_mn) / (fs_mx - fs_mn)
        fe = (fe_scr[i] - fe_mn) / (fe_mx - fe_mn)
        y = jnp.dot(h_scr[pl.ds(i * _BB, _BB)], w2_ref[:_HID],
                    preferred_element_type=jnp.float32)
        y_ref[pl.ds(i * _BB, _BB)] = (y + fs[:, None] * w2_ref[_HID][None, :]
                                      + fe[:, None] * w2_ref[_HID + 1][None, :])


def _lsm_body(p_ref, o_ref):
    z = p_ref[0] + p_ref[1]
    m = jnp.max(z, axis=1, keepdims=True)
    o_ref[...] = (z - m) - jnp.log(jnp.sum(jnp.exp(z - m), axis=1,
                                           keepdims=True))


def kernel(x, edge_index, W1, W2):
    row1 = edge_index[0].reshape(_NC, _NS, _NCHUNK1, _CHUNK1)
    col1 = edge_index[1].reshape(_NC, _NS, _NCHUNK1, _CHUNK1)
    pad_i = jnp.arange(_EP2 - _E, dtype=jnp.int32)
    pad_r = pad_i % _N
    pad_c = _N + pad_i % (_NP - _N)
    row2 = jnp.concatenate([edge_index[0], pad_r]).reshape(
        _NC, _NS, _NCHUNK2, _CHUNK2)
    col2 = jnp.concatenate([edge_index[1], pad_c]).reshape(
        _NC, _NS, _NCHUNK2, _CHUNK2)
    zeros1 = jnp.zeros((_RPT, _D1), jnp.float32)
    zeros2 = jnp.zeros((_RPT, _CLS), jnp.float32)

    table1 = pl.pallas_call(
        _table1_body,
        out_shape=jax.ShapeDtypeStruct((_N, _D1), jnp.float32),
    )(x)
    p1 = _sc_segsum(table1, row1, col1, zeros1, _D1, _CHUNK1, _NCHUNK1)
    y = pl.pallas_call(
        _layer1_body,
        out_shape=jax.ShapeDtypeStruct((_N, _CLS), jnp.float32),
        in_specs=[
            pl.BlockSpec(memory_space=pl.ANY),
            pl.BlockSpec(memory_space=pltpu.VMEM),
            pl.BlockSpec(memory_space=pltpu.VMEM),
        ],
        scratch_shapes=[
            pltpu.VMEM((2, _BB, _D1), jnp.float32),
            pltpu.VMEM((2, _BB, _D1), jnp.float32),
            pltpu.VMEM((_N, _HID), jnp.float32),
            pltpu.VMEM((_NB, _BB), jnp.float32),
            pltpu.VMEM((_NB, _BB), jnp.float32),
            pltpu.SemaphoreType.DMA,
            pltpu.SemaphoreType.DMA,
        ],
    )(p1, W1, W2)
    p2 = _sc_segsum(y, row2, col2, zeros2, _CLS, _CHUNK2, _NCHUNK2)
    return pl.pallas_call(
        _lsm_body,
        out_shape=jax.ShapeDtypeStruct((_N, _CLS), jnp.float32),
        grid=(_NB,),
        in_specs=[pl.BlockSpec((2, _BB, _CLS), lambda i: (0, i, 0))],
        out_specs=pl.BlockSpec((_BB, _CLS), lambda i: (i, 0)),
    )(p2)
